# baseline (speedup 1.0000x reference)
import functools

import jax
import jax.numpy as jnp
from jax import lax
from jax.experimental import pallas as pl
from jax.experimental.pallas import tpu as pltpu
from jax.experimental.pallas import tpu_sc as plsc

NUM_CLUSTERS = 8192
EMB_DIM = 256
L = 16
NC = 2
NS = 16
NW = NC * NS
R = NUM_CLUSTERS // NW
NCH = EMB_DIM // L
CHUNK = 64
NCHUNKS = R // CHUNK


def _tree_sum(xs):
    while len(xs) > 1:
        xs = [xs[j] + xs[j + 1] for j in range(0, len(xs), 2)]
    return xs[0]


def _router_body(z_hbm, cent_hbm, val_out, idx_out,
                 z_v, buf0, buf1, val_v, idx_v, sem0, sem1):
    c = lax.axis_index("c")
    s = lax.axis_index("s")
    wid = c * NS + s
    base = wid * R

    pltpu.sync_copy(z_hbm, z_v)

    bufs = (buf0, buf1)
    sems = (sem0, sem1)
    copies = {}
    copies[0] = pltpu.async_copy(
        cent_hbm.at[pl.ds(base, CHUNK), :], bufs[0], sems[0])

    zc = [z_v[pl.ds(k * L, L)] for k in range(NCH)]
    zsq_scan = plsc.cumsum(_tree_sum([z * z for z in zc]))

    neg_inf = jnp.full((L,), -jnp.inf, dtype=jnp.float32)
    zero_idx = jnp.zeros((L,), dtype=jnp.int32)
    eps = jnp.full((L,), 1e-16, dtype=jnp.float32)

    carry = (neg_inf, zero_idx, neg_inf, zero_idx)
    for ch in range(NCHUNKS):
        if ch + 1 < NCHUNKS:
            copies[ch + 1] = pltpu.async_copy(
                cent_hbm.at[pl.ds(base + (ch + 1) * CHUNK, CHUNK), :],
                bufs[(ch + 1) % 2], sems[(ch + 1) % 2])
        copies[ch].wait()
        buf = bufs[ch % 2]
        gbase = base + ch * CHUNK

        def row_t(r):
            prods, sqs = [], []
            for k in range(NCH):
                v = buf[r, pl.ds(k * L, L)]
                prods.append(v * zc[k])
                sqs.append(v * v)
            d = plsc.cumsum(_tree_sum(prods))
            sq = plsc.cumsum(_tree_sum(sqs))
            return d * jnp.abs(d) / jnp.maximum(zsq_scan * sq, eps)

        def pair_step(i, cy):
            vbA, viA, vbB, viB = cy
            r0 = i * 2
            tA = row_t(r0)
            tB = row_t(r0 + 1)
            mA = tA > vbA
            mB = tB > vbB
            iA = zero_idx + (gbase + r0)
            iB = zero_idx + (gbase + r0 + 1)
            return (jnp.where(mA, tA, vbA), jnp.where(mA, iA, viA),
                    jnp.where(mB, tB, vbB), jnp.where(mB, iB, viB))

        carry = lax.fori_loop(0, CHUNK // 2, pair_step, carry)

    vbA, viA, vbB, viB = carry
    take_b = (vbB > vbA) | ((vbB == vbA) & (viB < viA))
    vbest = jnp.where(take_b, vbB, vbA)
    vbidx = jnp.where(take_b, viB, viA)

    val_v[...] = vbest
    idx_v[...] = vbidx
    pltpu.sync_copy(val_v, val_out.at[wid])
    pltpu.sync_copy(idx_v, idx_out.at[wid])


_router = pl.kernel(
    _router_body,
    mesh=plsc.VectorSubcoreMesh(core_axis_name="c", subcore_axis_name="s"),
    compiler_params=pltpu.CompilerParams(needs_layout_passes=False),
    out_type=[
        jax.ShapeDtypeStruct((NW, L), jnp.float32),
        jax.ShapeDtypeStruct((NW, L), jnp.int32),
    ],
    scratch_types=[
        pltpu.VMEM((EMB_DIM,), jnp.float32),
        pltpu.VMEM((CHUNK, EMB_DIM), jnp.float32),
        pltpu.VMEM((CHUNK, EMB_DIM), jnp.float32),
        pltpu.VMEM((L,), jnp.float32),
        pltpu.VMEM((L,), jnp.int32),
        pltpu.SemaphoreType.DMA,
        pltpu.SemaphoreType.DMA,
    ],
)


@jax.jit
def kernel(z, centroids):
    vals, idxs = _router(z, centroids)
    t = jnp.argmax(vals[:, L - 1])
    return idxs[t, L - 1]
